# baseline (speedup 1.0000x reference)
import dataclasses
import functools

import jax
import jax.numpy as jnp
from jax import lax
from jax.experimental import pallas as pl
from jax.experimental.pallas import tpu as pltpu
from jax.experimental.pallas import tpu_sc as plsc

N_NODES = 10000
N_EDGES = 320000
D = 128

NC = 2
NS = 16
NW = NC * NS

N_PAD = 10240
E_PAD = 327680
EPW = E_PAD // NW
CHUNK = 80
NCHUNK = EPW // CHUNK
RPT = N_PAD // NS

_MESH = plsc.VectorSubcoreMesh(core_axis_name="c", subcore_axis_name="s")

_SC_PARAMS = pltpu.CompilerParams()
if "needs_layout_passes" in pltpu.CompilerParams.__dataclass_fields__:
    _SC_PARAMS = dataclasses.replace(_SC_PARAMS, needs_layout_passes=False)


@functools.partial(
    pl.kernel,
    out_type=jax.ShapeDtypeStruct((NC, N_PAD), jnp.float32),
    mesh=_MESH,
    compiler_params=_SC_PARAMS,
    scratch_types=[
        pltpu.VMEM((N_PAD,), jnp.float32),
        pltpu.VMEM((EPW,), jnp.int32),
        pltpu.VMEM((EPW,), jnp.float32),
        pltpu.VMEM((RPT,), jnp.float32),
        pltpu.VMEM((RPT,), jnp.float32),
        pltpu.VMEM_SHARED((NS, N_PAD), jnp.float32),
    ],
)
def _deg_call(dst_hbm, w_hbm, out_hbm, deg_v, dst_v, w_v, acc_v, tmp_v, stage):
    c = lax.axis_index("c")
    s = lax.axis_index("s")
    base = (s * NC + c) * EPW
    zeros = jnp.zeros((16,), jnp.float32)

    pltpu.sync_copy(dst_hbm.at[pl.ds(base, EPW)], dst_v)
    pltpu.sync_copy(w_hbm.at[pl.ds(base, EPW)], w_v)

    @pl.loop(0, N_PAD, step=16)
    def _(i):
        deg_v[pl.ds(i, 16)] = zeros

    @pl.loop(0, EPW, step=16)
    def _(g):
        plsc.addupdate_scatter(deg_v, [dst_v[pl.ds(g, 16)]], w_v[pl.ds(g, 16)])

    pltpu.sync_copy(deg_v, stage.at[s])
    plsc.subcore_barrier()

    col = s * RPT
    pltpu.sync_copy(stage.at[0, pl.ds(col, RPT)], acc_v)

    @pl.loop(1, NS)
    def _(r):
        pltpu.sync_copy(stage.at[r, pl.ds(col, RPT)], tmp_v)

        @pl.loop(0, RPT, step=16)
        def _(i):
            acc_v[pl.ds(i, 16)] = acc_v[pl.ds(i, 16)] + tmp_v[pl.ds(i, 16)]

    pltpu.sync_copy(acc_v, out_hbm.at[c, pl.ds(col, RPT)])


NBUF = 4
NSLOT = 8

NPT0 = 256
NPT1 = 256 - NPT0


@functools.partial(
    pl.kernel,
    out_type=jax.ShapeDtypeStruct((N_PAD, D), jnp.float32),
    mesh=_MESH,
    compiler_params=_SC_PARAMS,
    scratch_types=[
        pltpu.VMEM((NSLOT, CHUNK), jnp.int32),
        pltpu.VMEM((NSLOT, CHUNK), jnp.int32),
        pltpu.VMEM((NSLOT, CHUNK), jnp.float32),
        pltpu.VMEM((NBUF, CHUNK, D), jnp.float32)
    ] + [pltpu.SemaphoreType.DMA] * (2 * NBUF + NSLOT) + [
        pltpu.VMEM_SHARED((N_PAD, D), jnp.float32),
    ],
)
def _msg_call(g_hbm, src_hbm, dst_hbm, w_hbm, out_hbm,
              src_v, dst_v, w_v, rows_v, *rest):
    sgs = rest[0:NBUF]
    sss = rest[NBUF:2 * NBUF]
    sis = rest[2 * NBUF:2 * NBUF + NSLOT]
    acc = rest[-1]
    c = lax.axis_index("c")
    s = lax.axis_index("s")
    n = NPT0
    ebase = s * NPT0 * CHUNK
    zeros = jnp.zeros((16,), jnp.float32)

    def fire_i(sl, k):
        off = ebase + k * CHUNK
        pltpu.async_copy(src_hbm.at[pl.ds(off, CHUNK)], src_v.at[sl], sis[sl])
        pltpu.async_copy(dst_hbm.at[pl.ds(off, CHUNK)], dst_v.at[sl], sis[sl])
        pltpu.async_copy(w_hbm.at[pl.ds(off, CHUNK)], w_v.at[sl], sis[sl])

    def wait_i(sl, k):
        off = ebase + k * CHUNK
        pltpu.make_async_copy(src_hbm.at[pl.ds(off, CHUNK)], src_v.at[sl],
                              sis[sl]).wait()
        pltpu.make_async_copy(dst_hbm.at[pl.ds(off, CHUNK)], dst_v.at[sl],
                              sis[sl]).wait()
        pltpu.make_async_copy(w_hbm.at[pl.ds(off, CHUNK)], w_v.at[sl],
                              sis[sl]).wait()

    def fire_g(b, sl):
        pltpu.async_copy(g_hbm.at[src_v.at[sl]], rows_v.at[b], sgs[b])

    def wait_g(b, sl):
        pltpu.make_async_copy(g_hbm.at[src_v.at[sl]], rows_v.at[b],
                              sgs[b]).wait()

    def fire_s(b, sl):
        pltpu.async_copy(rows_v.at[b], acc.at[dst_v.at[sl]], sss[b], add=True)

    def wait_s(b, sl):
        pltpu.make_async_copy(rows_v.at[b], acc.at[dst_v.at[sl]], sss[b]).wait()

    @pl.when(c == 0)
    def _work():
        for j in range(NBUF):
            fire_i(j, j)
        wait_i(0, 0)
        fire_g(0, 0)
        wait_i(1, 1)
        fire_g(1, 1)

        @pl.loop(0, CHUNK)
        def _(r):
            for j in range(D // 16):
                rows_v[NBUF - 1, r, pl.ds(j * 16, 16)] = zeros

        @pl.loop(0, RPT // CHUNK)
        def _(i):
            pltpu.sync_copy(rows_v.at[NBUF - 1],
                            acc.at[pl.ds(s * RPT + i * CHUNK, CHUNK)])

        plsc.subcore_barrier()

        @pl.loop(0, n, step=NSLOT)
        def _(k8):
            for i in range(NSLOT):
                b = i % NBUF
                k = k8 + i
                wait_g(b, i)

                @pl.when(k >= 2)
                def _():
                    wait_s((i + 2) % NBUF, (i + NSLOT - 2) % NSLOT)

                @pl.when(k + 2 < n)
                def _():
                    wait_i((i + 2) % NSLOT, k + 2)
                    fire_g((i + 2) % NBUF, (i + 2) % NSLOT)

                @pl.when(k + NBUF < n)
                def _():
                    fire_i((i + NBUF) % NSLOT, k + NBUF)

                @pl.loop(0, CHUNK, step=16)
                def _(g):
                    w16 = w_v[i, pl.ds(g, 16)]
                    for r16 in range(16):
                        wv = jnp.full((16,), w16[r16])
                        for j in range(D // 16):
                            sl = pl.ds(j * 16, 16)
                            rows_v[b, g + r16, sl] = rows_v[b, g + r16, sl] * wv

                fire_s(b, i)

        wait_s(2, NSLOT - 2)
        wait_s(3, NSLOT - 1)
        plsc.subcore_barrier()
        pltpu.sync_copy(acc.at[pl.ds(s * RPT, RPT)],
                        out_hbm.at[pl.ds(s * RPT, RPT)])


_BLK = 1024
_GRID = N_PAD // _BLK


def _mm_body(x_ref, w_ref, o_ref):
    o_ref[...] = jnp.dot(x_ref[...], w_ref[...],
                         preferred_element_type=jnp.float32)


def _tc_matmul(x, w):
    return pl.pallas_call(
        _mm_body,
        grid=(_GRID,),
        in_specs=[pl.BlockSpec((_BLK, D), lambda i: (i, 0)),
                  pl.BlockSpec((D, D), lambda i: (0, 0))],
        out_specs=pl.BlockSpec((_BLK, D), lambda i: (i, 0)),
        out_shape=jax.ShapeDtypeStruct((N_PAD, D), jnp.float32),
    )(x, w)


def _dinv_body(degp_ref, h_ref, dinv_ref, g_ref):
    deg = degp_ref[0] + degp_ref[1] + 1.0
    dinv = jnp.where(deg > 0, lax.rsqrt(deg), 0.0)
    dinv_ref[...] = dinv
    g_ref[...] = h_ref[...] * dinv


def _tc_dinv_scale(degp, h):
    return pl.pallas_call(
        _dinv_body,
        grid=(_GRID,),
        in_specs=[pl.BlockSpec((NC, _BLK, 1), lambda i: (0, i, 0)),
                  pl.BlockSpec((_BLK, D), lambda i: (i, 0))],
        out_specs=[pl.BlockSpec((_BLK, 1), lambda i: (i, 0)),
                   pl.BlockSpec((_BLK, D), lambda i: (i, 0))],
        out_shape=[jax.ShapeDtypeStruct((N_PAD, 1), jnp.float32),
                   jax.ShapeDtypeStruct((N_PAD, D), jnp.float32)],
    )(degp, h)


def _layer_body(acc_ref, h_ref, dinv_ref, b_ref, w_ref, h2_ref, g2_ref):
    dinv = dinv_ref[...]
    z = dinv * acc_ref[...] + (dinv * dinv) * h_ref[...] + b_ref[...]
    z = jnp.maximum(z, 0.0)
    h2 = jnp.dot(z, w_ref[...], preferred_element_type=jnp.float32)
    h2_ref[...] = h2
    g2_ref[...] = h2 * dinv


def _tc_layer(acc, h, dinv, b, w):
    return pl.pallas_call(
        _layer_body,
        grid=(_GRID,),
        in_specs=[pl.BlockSpec((_BLK, D), lambda i: (i, 0)),
                  pl.BlockSpec((_BLK, D), lambda i: (i, 0)),
                  pl.BlockSpec((_BLK, 1), lambda i: (i, 0)),
                  pl.BlockSpec((1, D), lambda i: (0, 0)),
                  pl.BlockSpec((D, D), lambda i: (0, 0))],
        out_specs=[pl.BlockSpec((_BLK, D), lambda i: (i, 0)),
                   pl.BlockSpec((_BLK, D), lambda i: (i, 0))],
        out_shape=[jax.ShapeDtypeStruct((N_PAD, D), jnp.float32),
                   jax.ShapeDtypeStruct((N_PAD, D), jnp.float32)],
    )(acc, h, dinv, b, w)


def _final_body(acc_ref, h_ref, dinv_ref, b_ref, wo_ref, bo_ref, o_ref):
    dinv = dinv_ref[...]
    z = dinv * acc_ref[...] + (dinv * dinv) * h_ref[...] + b_ref[...]
    z = jnp.maximum(z, 0.0)
    o_ref[...] = jnp.dot(z, wo_ref[...],
                         preferred_element_type=jnp.float32) + bo_ref[...]


def _tc_final(acc, h, dinv, b, wo, bo):
    return pl.pallas_call(
        _final_body,
        grid=(_GRID,),
        in_specs=[pl.BlockSpec((_BLK, D), lambda i: (i, 0)),
                  pl.BlockSpec((_BLK, D), lambda i: (i, 0)),
                  pl.BlockSpec((_BLK, 1), lambda i: (i, 0)),
                  pl.BlockSpec((1, D), lambda i: (0, 0)),
                  pl.BlockSpec((D, 1), lambda i: (0, 0)),
                  pl.BlockSpec((1, 1), lambda i: (0, 0))],
        out_specs=pl.BlockSpec((_BLK, 1), lambda i: (i, 0)),
        out_shape=jax.ShapeDtypeStruct((N_PAD, 1), jnp.float32),
    )(acc, h, dinv, b, wo, bo)


def kernel(x, edge_index, edge_weight, W1, b1, W2, b2, Wo, bo):
    src = edge_index[0].astype(jnp.int32)
    dst = edge_index[1].astype(jnp.int32)
    w = edge_weight.astype(jnp.float32)

    pad_e = E_PAD - src.shape[0]
    pad_idx = jnp.full((pad_e,), N_PAD - 1, jnp.int32)
    src_p = jnp.concatenate([src, pad_idx])
    dst_p = jnp.concatenate([dst, pad_idx])
    w_p = jnp.concatenate([w, jnp.zeros((pad_e,), jnp.float32)])
    x_p = jnp.pad(x, ((0, N_PAD - x.shape[0]), (0, 0)))

    degp = _deg_call(dst_p, w_p)
    h1 = _tc_matmul(x_p, W1)
    dinv, g1 = _tc_dinv_scale(degp.reshape(NC, N_PAD, 1), h1)

    acc1 = _msg_call(g1, src_p, dst_p, w_p)
    h2, g2 = _tc_layer(acc1, h1, dinv, b1.reshape(1, D), W2)

    acc2 = _msg_call(g2, src_p, dst_p, w_p)
    out_p = _tc_final(acc2, h2, dinv, b2.reshape(1, D), Wo, bo.reshape(1, 1))

    return out_p[:N_NODES]


# speedup vs baseline: 1.4233x; 1.4233x over previous
import dataclasses
import functools

import jax
import jax.numpy as jnp
from jax import lax
from jax.experimental import pallas as pl
from jax.experimental.pallas import tpu as pltpu
from jax.experimental.pallas import tpu_sc as plsc

N_NODES = 10000
N_EDGES = 320000
D = 128

NC = 2
NS = 16
NW = NC * NS

N_PAD = 10240
E_PAD = 327680
EPW = E_PAD // NW
CHUNK = 80
NCHUNK = EPW // CHUNK
RPT = N_PAD // NS

_MESH = plsc.VectorSubcoreMesh(core_axis_name="c", subcore_axis_name="s")

_SC_PARAMS = pltpu.CompilerParams()
if "needs_layout_passes" in pltpu.CompilerParams.__dataclass_fields__:
    _SC_PARAMS = dataclasses.replace(_SC_PARAMS, needs_layout_passes=False)


@functools.partial(
    pl.kernel,
    out_type=jax.ShapeDtypeStruct((NC, N_PAD), jnp.float32),
    mesh=_MESH,
    compiler_params=_SC_PARAMS,
    scratch_types=[
        pltpu.VMEM((N_PAD,), jnp.float32),
        pltpu.VMEM((EPW,), jnp.int32),
        pltpu.VMEM((EPW,), jnp.float32),
        pltpu.VMEM((RPT,), jnp.float32),
        pltpu.VMEM((RPT,), jnp.float32),
        pltpu.VMEM_SHARED((NS, N_PAD), jnp.float32),
    ],
)
def _deg_call(dst_hbm, w_hbm, out_hbm, deg_v, dst_v, w_v, acc_v, tmp_v, stage):
    c = lax.axis_index("c")
    s = lax.axis_index("s")
    base = (s * NC + c) * EPW
    zeros = jnp.zeros((16,), jnp.float32)

    pltpu.sync_copy(dst_hbm.at[pl.ds(base, EPW)], dst_v)
    pltpu.sync_copy(w_hbm.at[pl.ds(base, EPW)], w_v)

    @pl.loop(0, N_PAD, step=16)
    def _(i):
        deg_v[pl.ds(i, 16)] = zeros

    @pl.loop(0, EPW, step=16)
    def _(g):
        plsc.addupdate_scatter(deg_v, [dst_v[pl.ds(g, 16)]], w_v[pl.ds(g, 16)])

    pltpu.sync_copy(deg_v, stage.at[s])
    plsc.subcore_barrier()

    col = s * RPT
    pltpu.sync_copy(stage.at[0, pl.ds(col, RPT)], acc_v)

    @pl.loop(1, NS)
    def _(r):
        pltpu.sync_copy(stage.at[r, pl.ds(col, RPT)], tmp_v)

        @pl.loop(0, RPT, step=16)
        def _(i):
            acc_v[pl.ds(i, 16)] = acc_v[pl.ds(i, 16)] + tmp_v[pl.ds(i, 16)]

    pltpu.sync_copy(acc_v, out_hbm.at[c, pl.ds(col, RPT)])


NBUF = 4
NSLOT = 8

NPT0 = 240
NPT1 = 256 - NPT0


@functools.partial(
    pl.kernel,
    out_type=jax.ShapeDtypeStruct((NC, N_PAD, D), jnp.float32),
    mesh=_MESH,
    compiler_params=_SC_PARAMS,
    scratch_types=[
        pltpu.VMEM((NSLOT, CHUNK), jnp.int32),
        pltpu.VMEM((NSLOT, CHUNK), jnp.int32),
        pltpu.VMEM((NSLOT, CHUNK), jnp.float32),
        pltpu.VMEM((NBUF, CHUNK, D), jnp.float32)
    ] + [pltpu.SemaphoreType.DMA] * (2 * NBUF + NSLOT) + [
        pltpu.VMEM_SHARED((N_PAD, D), jnp.float32),
    ],
)
def _msg_call(g_hbm, src_hbm, dst_hbm, w_hbm, out_hbm,
              src_v, dst_v, w_v, rows_v, *rest):
    sgs = rest[0:NBUF]
    sss = rest[NBUF:2 * NBUF]
    sis = rest[2 * NBUF:2 * NBUF + NSLOT]
    acc = rest[-1]
    c = lax.axis_index("c")
    s = lax.axis_index("s")
    n = jnp.where(c == 0, NPT0, NPT1)
    ebase = jnp.where(c == 0, s * NPT0, NS * NPT0 + s * NPT1) * CHUNK
    zeros = jnp.zeros((16,), jnp.float32)

    def fire_i(sl, k):
        off = ebase + k * CHUNK
        pltpu.async_copy(src_hbm.at[pl.ds(off, CHUNK)], src_v.at[sl], sis[sl])
        pltpu.async_copy(dst_hbm.at[pl.ds(off, CHUNK)], dst_v.at[sl], sis[sl])
        pltpu.async_copy(w_hbm.at[pl.ds(off, CHUNK)], w_v.at[sl], sis[sl])

    def wait_i(sl, k):
        off = ebase + k * CHUNK
        pltpu.make_async_copy(src_hbm.at[pl.ds(off, CHUNK)], src_v.at[sl],
                              sis[sl]).wait()
        pltpu.make_async_copy(dst_hbm.at[pl.ds(off, CHUNK)], dst_v.at[sl],
                              sis[sl]).wait()
        pltpu.make_async_copy(w_hbm.at[pl.ds(off, CHUNK)], w_v.at[sl],
                              sis[sl]).wait()

    def fire_g(b, sl):
        pltpu.async_copy(g_hbm.at[src_v.at[sl]], rows_v.at[b], sgs[b])

    def wait_g(b, sl):
        pltpu.make_async_copy(g_hbm.at[src_v.at[sl]], rows_v.at[b],
                              sgs[b]).wait()

    def fire_s(b, sl):
        pltpu.async_copy(rows_v.at[b], acc.at[dst_v.at[sl]], sss[b], add=True)

    def wait_s(b, sl):
        pltpu.make_async_copy(rows_v.at[b], acc.at[dst_v.at[sl]], sss[b]).wait()

    def _work():
        for j in range(NBUF):
            fire_i(j, j)
        wait_i(0, 0)
        fire_g(0, 0)
        wait_i(1, 1)
        fire_g(1, 1)

        @pl.loop(0, CHUNK)
        def _(r):
            for j in range(D // 16):
                rows_v[NBUF - 1, r, pl.ds(j * 16, 16)] = zeros

        @pl.loop(0, RPT // CHUNK)
        def _(i):
            pltpu.sync_copy(rows_v.at[NBUF - 1],
                            acc.at[pl.ds(s * RPT + i * CHUNK, CHUNK)])

        plsc.subcore_barrier()

        @pl.loop(0, n, step=NSLOT)
        def _(k8):
            for i in range(NSLOT):
                b = i % NBUF
                k = k8 + i
                wait_g(b, i)

                @pl.when(k >= 2)
                def _():
                    wait_s((i + 2) % NBUF, (i + NSLOT - 2) % NSLOT)

                @pl.when(k + 2 < n)
                def _():
                    wait_i((i + 2) % NSLOT, k + 2)
                    fire_g((i + 2) % NBUF, (i + 2) % NSLOT)

                @pl.when(k + NBUF < n)
                def _():
                    fire_i((i + NBUF) % NSLOT, k + NBUF)

                @pl.loop(0, CHUNK, step=16)
                def _(g):
                    w16 = w_v[i, pl.ds(g, 16)]
                    for r16 in range(16):
                        wv = jnp.full((16,), w16[r16])
                        for j in range(D // 16):
                            sl = pl.ds(j * 16, 16)
                            rows_v[b, g + r16, sl] = rows_v[b, g + r16, sl] * wv

                fire_s(b, i)

        wait_s(2, NSLOT - 2)
        wait_s(3, NSLOT - 1)
        plsc.subcore_barrier()
        pltpu.sync_copy(acc.at[pl.ds(s * RPT, RPT)],
                        out_hbm.at[c, pl.ds(s * RPT, RPT)])

    _work()


_BLK = 1024
_GRID = N_PAD // _BLK


def _mm_body(x_ref, w_ref, o_ref):
    o_ref[...] = jnp.dot(x_ref[...], w_ref[...],
                         preferred_element_type=jnp.float32)


def _tc_matmul(x, w):
    return pl.pallas_call(
        _mm_body,
        grid=(_GRID,),
        in_specs=[pl.BlockSpec((_BLK, D), lambda i: (i, 0)),
                  pl.BlockSpec((D, D), lambda i: (0, 0))],
        out_specs=pl.BlockSpec((_BLK, D), lambda i: (i, 0)),
        out_shape=jax.ShapeDtypeStruct((N_PAD, D), jnp.float32),
    )(x, w)


def _dinv_body(degp_ref, h_ref, dinv_ref, g_ref):
    deg = degp_ref[0] + degp_ref[1] + 1.0
    dinv = jnp.where(deg > 0, lax.rsqrt(deg), 0.0)
    dinv_ref[...] = dinv
    g_ref[...] = h_ref[...] * dinv


def _tc_dinv_scale(degp, h):
    return pl.pallas_call(
        _dinv_body,
        grid=(_GRID,),
        in_specs=[pl.BlockSpec((NC, _BLK, 1), lambda i: (0, i, 0)),
                  pl.BlockSpec((_BLK, D), lambda i: (i, 0))],
        out_specs=[pl.BlockSpec((_BLK, 1), lambda i: (i, 0)),
                   pl.BlockSpec((_BLK, D), lambda i: (i, 0))],
        out_shape=[jax.ShapeDtypeStruct((N_PAD, 1), jnp.float32),
                   jax.ShapeDtypeStruct((N_PAD, D), jnp.float32)],
    )(degp, h)


def _layer_body(acc_ref, h_ref, dinv_ref, b_ref, w_ref, h2_ref, g2_ref):
    dinv = dinv_ref[...]
    z = dinv * (acc_ref[0] + acc_ref[1]) + (dinv * dinv) * h_ref[...] + b_ref[...]
    z = jnp.maximum(z, 0.0)
    h2 = jnp.dot(z, w_ref[...], preferred_element_type=jnp.float32)
    h2_ref[...] = h2
    g2_ref[...] = h2 * dinv


def _tc_layer(acc, h, dinv, b, w):
    return pl.pallas_call(
        _layer_body,
        grid=(_GRID,),
        in_specs=[pl.BlockSpec((NC, _BLK, D), lambda i: (0, i, 0)),
                  pl.BlockSpec((_BLK, D), lambda i: (i, 0)),
                  pl.BlockSpec((_BLK, 1), lambda i: (i, 0)),
                  pl.BlockSpec((1, D), lambda i: (0, 0)),
                  pl.BlockSpec((D, D), lambda i: (0, 0))],
        out_specs=[pl.BlockSpec((_BLK, D), lambda i: (i, 0)),
                   pl.BlockSpec((_BLK, D), lambda i: (i, 0))],
        out_shape=[jax.ShapeDtypeStruct((N_PAD, D), jnp.float32),
                   jax.ShapeDtypeStruct((N_PAD, D), jnp.float32)],
    )(acc, h, dinv, b, w)


def _final_body(acc_ref, h_ref, dinv_ref, b_ref, wo_ref, bo_ref, o_ref):
    dinv = dinv_ref[...]
    z = dinv * (acc_ref[0] + acc_ref[1]) + (dinv * dinv) * h_ref[...] + b_ref[...]
    z = jnp.maximum(z, 0.0)
    o_ref[...] = jnp.dot(z, wo_ref[...],
                         preferred_element_type=jnp.float32) + bo_ref[...]


def _tc_final(acc, h, dinv, b, wo, bo):
    return pl.pallas_call(
        _final_body,
        grid=(_GRID,),
        in_specs=[pl.BlockSpec((NC, _BLK, D), lambda i: (0, i, 0)),
                  pl.BlockSpec((_BLK, D), lambda i: (i, 0)),
                  pl.BlockSpec((_BLK, 1), lambda i: (i, 0)),
                  pl.BlockSpec((1, D), lambda i: (0, 0)),
                  pl.BlockSpec((D, 1), lambda i: (0, 0)),
                  pl.BlockSpec((1, 1), lambda i: (0, 0))],
        out_specs=pl.BlockSpec((_BLK, 1), lambda i: (i, 0)),
        out_shape=jax.ShapeDtypeStruct((N_PAD, 1), jnp.float32),
    )(acc, h, dinv, b, wo, bo)


def kernel(x, edge_index, edge_weight, W1, b1, W2, b2, Wo, bo):
    src = edge_index[0].astype(jnp.int32)
    dst = edge_index[1].astype(jnp.int32)
    w = edge_weight.astype(jnp.float32)

    pad_e = E_PAD - src.shape[0]
    pad_idx = jnp.full((pad_e,), N_PAD - 1, jnp.int32)
    src_p = jnp.concatenate([src, pad_idx])
    dst_p = jnp.concatenate([dst, pad_idx])
    w_p = jnp.concatenate([w, jnp.zeros((pad_e,), jnp.float32)])
    x_p = jnp.pad(x, ((0, N_PAD - x.shape[0]), (0, 0)))

    degp = _deg_call(dst_p, w_p)
    h1 = _tc_matmul(x_p, W1)
    dinv, g1 = _tc_dinv_scale(degp.reshape(NC, N_PAD, 1), h1)

    acc1 = _msg_call(g1, src_p, dst_p, w_p)
    h2, g2 = _tc_layer(acc1, h1, dinv, b1.reshape(1, D), W2)

    acc2 = _msg_call(g2, src_p, dst_p, w_p)
    out_p = _tc_final(acc2, h2, dinv, b2.reshape(1, D), Wo, bo.reshape(1, 1))

    return out_p[:N_NODES]
